# baseline (speedup 1.0000x reference)
import functools

import jax
import jax.numpy as jnp
from jax import lax
from jax.experimental import pallas as pl
from jax.experimental.pallas import tpu as pltpu
from jax.experimental.pallas import tpu_sc as plsc

NC = 2
NS = 16
NW = NC * NS
CH = 128


def _ceil_div(a, b):
    return -(-a // b)


def _encoder_body(x_ref, w_ref, b_ref, g_ref, be_ref, h_ref):
    y = jnp.dot(x_ref[...], w_ref[...], preferred_element_type=jnp.float32)
    y = y + b_ref[...][None, :]
    m = jnp.mean(y, axis=0)
    v = jnp.mean((y - m[None, :]) ** 2, axis=0)
    y = (y - m[None, :]) * lax.rsqrt(v + 1e-5) * g_ref[...][None, :] + be_ref[...][None, :]
    h_ref[...] = jnp.maximum(y, 0.0)


def _loop_stats_sc(dst, edge_attr, n_pad):
    E = dst.shape[0]
    T = _ceil_div(_ceil_div(E, NW), CH)
    per_w = T * CH
    EP = per_w * NW
    dstp = jnp.zeros((EP,), jnp.int32).at[:E].set(dst).reshape(NW, T, CH)
    attrp = jnp.zeros((EP, 32), jnp.float32)
    attrp = attrp.at[:E, :16].set(edge_attr).at[:E, 16].set(1.0)
    attrp = attrp.reshape(NW, per_w, 32)
    stripe = n_pad // NS

    mesh = plsc.VectorSubcoreMesh(core_axis_name="c", subcore_axis_name="s")

    @functools.partial(
        pl.kernel,
        out_type=jax.ShapeDtypeStruct((NC, n_pad, 32), jnp.float32),
        mesh=mesh,
        compiler_params=pltpu.CompilerParams(needs_layout_passes=False, use_tc_tiling_on_sc=False),
        scratch_types=[
            pltpu.VMEM((T, CH), jnp.int32),
            pltpu.VMEM((CH, 32), jnp.float32),
            pltpu.VMEM((CH, 32), jnp.float32),
            pltpu.VMEM_SHARED((n_pad, 32), jnp.float32),
        ],
    )
    def k(dst_hbm, attr_hbm, ssum_hbm, dst_vm, attr_vm, zero_vm, s_sh):
        core = lax.axis_index("c")
        sub = lax.axis_index("s")
        wid = sub * NC + core
        zeros16 = jnp.zeros((16,), jnp.float32)
        pltpu.sync_copy(dst_hbm.at[wid], dst_vm)

        def z_zero(i, carry):
            zero_vm[i, pl.ds(0, 16)] = zeros16
            zero_vm[i, pl.ds(16, 16)] = zeros16
            return carry

        lax.fori_loop(0, CH, z_zero, 0)

        base = sub * stripe
        for off in range(0, stripe, CH):
            pltpu.sync_copy(zero_vm, s_sh.at[pl.ds(base + off, CH)])
        plsc.subcore_barrier()

        def chunk(c, carry):
            pltpu.sync_copy(attr_hbm.at[wid, pl.ds(c * CH, CH)], attr_vm)
            pltpu.sync_copy(attr_vm, s_sh.at[dst_vm.at[c]], add=True)
            return carry

        lax.fori_loop(0, T, chunk, 0)

        plsc.subcore_barrier()
        pltpu.sync_copy(s_sh.at[pl.ds(base, stripe)],
                        ssum_hbm.at[core, pl.ds(base, stripe)])

    return k(dstp, attrp)


XW = 132


def _edge_pass_sc(idxp, a_d, xse, n_pad, E2):
    n = a_d.shape[0]
    T = idxp.shape[1]
    per_w = T * CH
    stripe_m = n // NS
    stripe_d = n_pad // NS

    mesh = plsc.VectorSubcoreMesh(core_axis_name="c", subcore_axis_name="s")

    @functools.partial(
        pl.kernel,
        out_type=(jax.ShapeDtypeStruct((NC, n, XW), jnp.float32),
                  jax.ShapeDtypeStruct((NC, n_pad), jnp.float32)),
        mesh=mesh,
        compiler_params=pltpu.CompilerParams(needs_layout_passes=False, use_tc_tiling_on_sc=False),
        scratch_types=[
            pltpu.VMEM((2, 3, CH), jnp.int32),
            pltpu.VMEM((n,), jnp.float32),
            pltpu.VMEM((2, CH, XW), jnp.float32),
            pltpu.VMEM((CH,), jnp.int32),
            pltpu.VMEM((CH + 16,), jnp.float32),
            pltpu.VMEM_SHARED((n, XW), jnp.float32),
            pltpu.VMEM_SHARED((n_pad,), jnp.float32),
            pltpu.SemaphoreType.DMA,
            pltpu.SemaphoreType.DMA,
            pltpu.SemaphoreType.DMA,
            pltpu.SemaphoreType.DMA,
        ],
    )
    def k(idx_hbm, ad_hbm, xs_hbm, m_hbm, den_hbm,
          idx_vm, ad_vm, rows_vm, dst_vm, s_vm, m_sh, d_sh,
          sg0, sg1, si0, si1):
        sg = (sg0, sg1)
        si = (si0, si1)
        core = lax.axis_index("c")
        sub = lax.axis_index("s")
        wid = sub * NC + core
        zeros16 = jnp.zeros((16,), jnp.float32)
        col128 = jnp.full((16,), 128, jnp.int32)
        pltpu.sync_copy(ad_hbm, ad_vm)

        def z_rows(i, carry):
            for j in range(8):
                rows_vm[0, i, pl.ds(j * 16, 16)] = zeros16
            rows_vm[0, i, pl.ds(XW - 16, 16)] = zeros16
            return carry

        lax.fori_loop(0, CH, z_rows, 0)
        for j in range(9):
            s_vm[pl.ds(j * 16, 16)] = zeros16

        base_m = sub * stripe_m
        for off in range(0, stripe_m, CH):
            blk = min(CH, stripe_m - off)
            pltpu.sync_copy(rows_vm.at[0, pl.ds(0, blk)],
                            m_sh.at[pl.ds(base_m + off, blk)])
        base_d = sub * stripe_d
        for off in range(0, stripe_d, CH):
            pltpu.sync_copy(s_vm.at[pl.ds(0, CH)], d_sh.at[pl.ds(base_d + off, CH)])
        plsc.subcore_barrier()

        def start_idx(buf, c):
            pltpu.async_copy(idx_hbm.at[wid, c], idx_vm.at[buf], si[buf])

        def wait_idx(buf, c):
            pltpu.make_async_copy(idx_hbm.at[wid, c], idx_vm.at[buf], si[buf]).wait()

        def start_gather(buf):
            pltpu.async_copy(xs_hbm.at[idx_vm.at[buf, 0]], rows_vm.at[buf], sg[buf])

        def wait_gather(buf, c):
            pltpu.make_async_copy(xs_hbm.at[idx_vm.at[buf, 0]], rows_vm.at[buf],
                                  sg[buf]).wait()

        start_idx(0, 0)
        wait_idx(0, 0)
        start_gather(0)
        start_idx(1, 1)

        def body(t, carry):
            for b in (0, 1):
                c = 2 * t + b
                nb = 1 - b
                for v in range(8):
                    didx = idx_vm[b, 1, pl.ds(v * 16, 16)]
                    ae = plsc.bitcast(idx_vm[b, 2, pl.ds(v * 16, 16)], jnp.float32)
                    al = plsc.load_gather(ad_vm, [didx]) + ae
                    s_vm[pl.ds(v * 16, 16)] = al
                    dst_vm[pl.ds(v * 16, 16)] = didx
                @pl.when(c + 1 < T)
                def _():
                    wait_idx(nb, c + 1)
                    start_gather(nb)
                wait_gather(b, c)
                @pl.when(c + 2 < T)
                def _():
                    start_idx(b, c + 2)
                for v in range(8):
                    ridx = lax.iota(jnp.int32, 16) + v * 16
                    asv = plsc.load_gather(rows_vm.at[b], [ridx, col128])
                    al = s_vm[pl.ds(v * 16, 16)] + asv
                    al = jnp.maximum(al, 0.2 * al)
                    s = jnp.exp(al)
                    gid = wid * per_w + c * CH + v * 16 + lax.iota(jnp.int32, 16)
                    s = jnp.where(gid < E2, s, 0.0)
                    s_vm[pl.ds(v * 16, 16)] = s

                def scale(i, carry2):
                    sc = s_vm[pl.ds(i, 16)][0]
                    for j in range(8):
                        rows_vm[b, i, pl.ds(j * 16, 16)] = (
                            rows_vm[b, i, pl.ds(j * 16, 16)] * sc)
                    return carry2

                lax.fori_loop(0, CH, scale, 0)
                pltpu.sync_copy(rows_vm.at[b], m_sh.at[dst_vm], add=True)
                pltpu.sync_copy(s_vm.at[pl.ds(0, CH)], d_sh.at[dst_vm], add=True)
            return carry

        lax.fori_loop(0, T // 2, body, 0)

        plsc.subcore_barrier()
        pltpu.sync_copy(m_sh.at[pl.ds(base_m, stripe_m)],
                        m_hbm.at[core, pl.ds(base_m, stripe_m)])
        pltpu.sync_copy(d_sh.at[pl.ds(base_d, stripe_d)],
                        den_hbm.at[core, pl.ds(base_d, stripe_d)])

    return k(idxp, a_d, xse)


def _dense_body(h_ref, w_ref, asrc_ref, adst_ref, xse_ref, ad_ref):
    n = h_ref.shape[0]
    xs = jnp.dot(h_ref[...], w_ref[...], preferred_element_type=jnp.float32)
    a_s = xs @ asrc_ref[...]
    xse_ref[...] = jnp.concatenate(
        [xs, a_s[:, None], jnp.zeros((n, XW - 129), jnp.float32)], axis=1)
    ad_ref[...] = xs @ adst_ref[...]


def _post_body(m_ref, den_ref, bias_ref, bng_ref, bnb_ref, h_ref, hout_ref):
    n = h_ref.shape[0]
    msum = m_ref[0] + m_ref[1]
    den = den_ref[0] + den_ref[1]
    out = msum[:, :128] / (den[:n, None] + 1e-16) + bias_ref[...][None, :]
    m = jnp.mean(out, axis=0)
    v = jnp.mean((out - m[None, :]) ** 2, axis=0)
    out = (out - m[None, :]) * lax.rsqrt(v + 1e-5) * bng_ref[...][None, :] + bnb_ref[...][None, :]
    hout_ref[...] = h_ref[...] + jnp.maximum(out, 0.0)


def _head_body(h_ref, w_ref, b_ref, o_ref):
    g = jnp.mean(h_ref[...], axis=0, keepdims=True)
    o_ref[...] = jnp.maximum(
        jnp.dot(g, w_ref[...], preferred_element_type=jnp.float32) + b_ref[...][None, :],
        0.0)


def _ae_real_body(attr_r_ref, we_ref, ae_ref, out_ref):
    L = we_ref.shape[0]
    w_all = jnp.sum(we_ref[...] * ae_ref[...][:, None, :], axis=2)
    eye8 = (lax.broadcasted_iota(jnp.int32, (8, 8), 0) ==
            lax.broadcasted_iota(jnp.int32, (8, 8), 1)).astype(jnp.float32)
    wbig = (eye8[:, None, :, None] *
            jnp.transpose(w_all)[None, :, None, :]).reshape(128, 8 * L)
    out_ref[...] = jnp.dot(attr_r_ref[...], wbig,
                           preferred_element_type=jnp.float32)


def _loop_ae_body(ssum_ref, we_ref, ae_ref, loopae_ref):
    w_all = jnp.sum(we_ref[...] * ae_ref[...][:, None, :], axis=2)
    n = loopae_ref.shape[0]
    ssum = ssum_ref[0] + ssum_ref[1]
    attr_sum = ssum[:n, :16]
    deg = ssum[:n, 16]
    inv = 1.0 / jnp.maximum(deg, 1.0)
    loopae_ref[...] = jnp.dot(attr_sum, jnp.transpose(w_all),
                              preferred_element_type=jnp.float32) * inv[:, None]


def kernel(x, edge_index, edge_attr, enc_W, enc_b, enc_gamma, enc_beta, W,
           att_src, att_dst, W_edge, att_edge, bias, bn_gamma, bn_beta, out_W, out_b):
    n = x.shape[0]
    E = edge_index.shape[1]
    L = W.shape[0]
    src, dst = edge_index[0], edge_index[1]

    h = pl.pallas_call(
        _encoder_body,
        out_shape=jax.ShapeDtypeStruct((n, enc_W.shape[1]), jnp.float32),
    )(x, enc_W, enc_b, enc_gamma, enc_beta)

    n_pad = NS * CH * _ceil_div(n, NS * CH)
    ssum = _loop_stats_sc(dst, edge_attr, n_pad)

    attr_r = edge_attr.reshape(E // 8, 128)
    n_blk = 8
    br = E // 8 // n_blk
    a_real = pl.pallas_call(
        _ae_real_body,
        grid=(n_blk,),
        in_specs=[pl.BlockSpec((br, 128), lambda i: (i, 0)),
                  pl.BlockSpec(W_edge.shape, lambda i: (0, 0, 0)),
                  pl.BlockSpec(att_edge.shape, lambda i: (0, 0))],
        out_specs=pl.BlockSpec((br, 8 * L), lambda i: (i, 0)),
        out_shape=jax.ShapeDtypeStruct((E // 8, 8 * L), jnp.float32),
    )(attr_r, W_edge, att_edge).reshape(E, L)
    loop_ae = pl.pallas_call(
        _loop_ae_body,
        out_shape=jax.ShapeDtypeStruct((n, L), jnp.float32),
    )(ssum, W_edge, att_edge)

    loop = jnp.arange(n, dtype=src.dtype)
    src2 = jnp.concatenate([src, loop])
    dst2 = jnp.concatenate([dst, loop])
    ae2 = jnp.concatenate([a_real, loop_ae], axis=0)

    E2 = E + n
    T = 2 * _ceil_div(_ceil_div(_ceil_div(E2, NW), CH), 2)
    per_w = T * CH
    E2P = per_w * NW
    srcp = jnp.zeros((E2P,), jnp.int32).at[:E2].set(src2).reshape(NW, T, CH)
    dstp = jnp.zeros((E2P,), jnp.int32).at[:E2].set(dst2).reshape(NW, T, CH)
    aep = jnp.zeros((E2P, L), jnp.float32).at[:E2].set(ae2)
    ae_bits = lax.bitcast_convert_type(aep, jnp.int32).reshape(NW, T, CH, L)
    idxp = [jnp.stack([srcp, dstp, ae_bits[..., l]], axis=2) for l in range(L)]

    for l in range(L):
        xse, a_d = pl.pallas_call(
            _dense_body,
            out_shape=(jax.ShapeDtypeStruct((n, XW), jnp.float32),
                       jax.ShapeDtypeStruct((n,), jnp.float32)),
        )(h, W[l], att_src[l], att_dst[l])
        m_parts, den_parts = _edge_pass_sc(idxp[l], a_d, xse, n_pad, E2)
        h = pl.pallas_call(
            _post_body,
            out_shape=jax.ShapeDtypeStruct((n, W.shape[2]), jnp.float32),
        )(m_parts, den_parts, bias[l], bn_gamma[l], bn_beta[l], h)

    return pl.pallas_call(
        _head_body,
        out_shape=jax.ShapeDtypeStruct((1, out_W.shape[1]), jnp.float32),
    )(h, out_W, out_b)


# speedup vs baseline: 1.8130x; 1.8130x over previous
import functools

import jax
import jax.numpy as jnp
from jax import lax
from jax.experimental import pallas as pl
from jax.experimental.pallas import tpu as pltpu
from jax.experimental.pallas import tpu_sc as plsc

NC = 2
NS = 16
NW = NC * NS
CH = 128


def _ceil_div(a, b):
    return -(-a // b)


def _encoder_body(x_ref, w_ref, b_ref, g_ref, be_ref, h_ref):
    y = jnp.dot(x_ref[...], w_ref[...], preferred_element_type=jnp.float32)
    y = y + b_ref[...][None, :]
    m = jnp.mean(y, axis=0)
    v = jnp.mean((y - m[None, :]) ** 2, axis=0)
    y = (y - m[None, :]) * lax.rsqrt(v + 1e-5) * g_ref[...][None, :] + be_ref[...][None, :]
    h_ref[...] = jnp.maximum(y, 0.0)


def _loop_stats_sc(edge_index, edge_attr, n, n_pad):
    E = edge_attr.shape[0]
    n_chunks = E // CH
    full = n_chunks // NW
    extra = n_chunks - full * NW
    stripe_s = n // NS
    stripe_d = n_pad // NS

    mesh = plsc.VectorSubcoreMesh(core_axis_name="c", subcore_axis_name="s")

    @functools.partial(
        pl.kernel,
        out_type=(jax.ShapeDtypeStruct((NC, n, 16), jnp.float32),
                  jax.ShapeDtypeStruct((NC, n_pad), jnp.float32)),
        mesh=mesh,
        compiler_params=pltpu.CompilerParams(needs_layout_passes=False, use_tc_tiling_on_sc=False),
        scratch_types=[
            pltpu.VMEM((CH,), jnp.int32),
            pltpu.VMEM((CH, 16), jnp.float32),
            pltpu.VMEM((CH,), jnp.float32),
            pltpu.VMEM_SHARED((n, 16), jnp.float32),
            pltpu.VMEM_SHARED((n_pad,), jnp.float32),
        ],
    )
    def k(ei_hbm, attr_hbm, ssum_hbm, deg_hbm, dst_vm, attr_vm, ones_vm, s_sh, d_sh):
        core = lax.axis_index("c")
        sub = lax.axis_index("s")
        wid = sub * NC + core
        zeros16 = jnp.zeros((16,), jnp.float32)

        def z_attr(i, carry):
            attr_vm[i, pl.ds(0, 16)] = zeros16
            return carry

        lax.fori_loop(0, CH, z_attr, 0)
        for j in range(CH // 16):
            ones_vm[pl.ds(j * 16, 16)] = zeros16

        base_s = sub * stripe_s
        for off in range(0, stripe_s, CH):
            blk = min(CH, stripe_s - off)
            pltpu.sync_copy(attr_vm.at[pl.ds(0, blk)],
                            s_sh.at[pl.ds(base_s + off, blk)])
        base_d = sub * stripe_d
        for off in range(0, stripe_d, CH):
            pltpu.sync_copy(ones_vm, d_sh.at[pl.ds(base_d + off, CH)])
        plsc.subcore_barrier()

        ones16 = jnp.ones((16,), jnp.float32)
        for j in range(CH // 16):
            ones_vm[pl.ds(j * 16, 16)] = ones16

        t_w = full + jnp.where(wid < extra, 1, 0)

        def chunk(t, carry):
            cid = wid + NW * t
            off = cid * CH
            pltpu.sync_copy(ei_hbm.at[1, pl.ds(off, CH)], dst_vm)
            pltpu.sync_copy(attr_hbm.at[pl.ds(off, CH)], attr_vm)
            pltpu.sync_copy(attr_vm, s_sh.at[dst_vm], add=True)
            pltpu.sync_copy(ones_vm, d_sh.at[dst_vm], add=True)
            return carry

        lax.fori_loop(0, t_w, chunk, 0)

        plsc.subcore_barrier()
        pltpu.sync_copy(s_sh.at[pl.ds(base_s, stripe_s)],
                        ssum_hbm.at[core, pl.ds(base_s, stripe_s)])
        pltpu.sync_copy(d_sh.at[pl.ds(base_d, stripe_d)],
                        deg_hbm.at[core, pl.ds(base_d, stripe_d)])

    return k(edge_index, edge_attr)


XW = 132


def _edge_pass_emu(edge_index, ae_e, loop_ae_l, a_d, xse, n_pad):
    n = a_d.shape[0]
    E = ae_e.shape[0]
    src, dst = edge_index[0], edge_index[1]
    a_s = xse[:, 128]
    xs = xse[:, :128]
    al = a_s[src] + a_d[dst] + ae_e
    al = jnp.where(al > 0, al, 0.2 * al)
    s = jnp.exp(al)
    nodes = jnp.arange(n)
    al2 = a_s + a_d + loop_ae_l
    al2 = jnp.where(al2 > 0, al2, 0.2 * al2)
    s2 = jnp.exp(al2)
    m = jax.ops.segment_sum(xs[src] * s[:, None], dst, num_segments=n) + xs * s2[:, None]
    m = jnp.concatenate([m, jnp.zeros((n, XW - 128))], axis=1)
    d = jax.ops.segment_sum(s, dst, num_segments=n_pad) + \
        jnp.concatenate([s2, jnp.zeros((n_pad - n,))])
    return jnp.stack([m, jnp.zeros_like(m)]), jnp.stack([d, jnp.zeros_like(d)])


def _edge_pass_sc(edge_index, ae_e, loop_ae_l, a_d, xse, n_pad):
    n = a_d.shape[0]
    E = ae_e.shape[0]
    n_chunks = E // CH
    full = n_chunks // NW
    full -= full % 2
    extra = n_chunks - full * NW
    ex_pairs = extra // 2
    s_chunks = _ceil_div(n, CH)
    stripe_m = n // NS
    stripe_d = n_pad // NS

    mesh = plsc.VectorSubcoreMesh(core_axis_name="c", subcore_axis_name="s")

    @functools.partial(
        pl.kernel,
        out_type=(jax.ShapeDtypeStruct((NC, n, XW), jnp.float32),
                  jax.ShapeDtypeStruct((NC, n_pad), jnp.float32)),
        mesh=mesh,
        compiler_params=pltpu.CompilerParams(needs_layout_passes=False, use_tc_tiling_on_sc=False),
        scratch_types=[
            pltpu.VMEM((2, 2, CH), jnp.int32),
            pltpu.VMEM((2, CH), jnp.float32),
            pltpu.VMEM((n,), jnp.float32),
            pltpu.VMEM((2, CH, XW), jnp.float32),
            pltpu.VMEM((CH,), jnp.int32),
            pltpu.VMEM((CH + 16,), jnp.float32),
            pltpu.VMEM_SHARED((n, XW), jnp.float32),
            pltpu.VMEM_SHARED((n_pad,), jnp.float32),
            pltpu.SemaphoreType.DMA,
            pltpu.SemaphoreType.DMA,
            pltpu.SemaphoreType.DMA,
            pltpu.SemaphoreType.DMA,
        ],
    )
    def k(ei_hbm, ae_hbm, lae_hbm, ad_hbm, xs_hbm, m_hbm, den_hbm,
          idx_vm, ae_vm, ad_vm, rows_vm, dst_vm, s_vm, m_sh, d_sh,
          sg0, sg1, si0, si1):
        sg = (sg0, sg1)
        si = (si0, si1)
        core = lax.axis_index("c")
        sub = lax.axis_index("s")
        wid = sub * NC + core
        zeros16 = jnp.zeros((16,), jnp.float32)
        col128 = jnp.full((16,), 128, jnp.int32)
        pltpu.sync_copy(ad_hbm, ad_vm)

        def z_rows(i, carry):
            for j in range(8):
                rows_vm[0, i, pl.ds(j * 16, 16)] = zeros16
            rows_vm[0, i, pl.ds(XW - 16, 16)] = zeros16
            return carry

        lax.fori_loop(0, CH, z_rows, 0)
        for j in range(9):
            s_vm[pl.ds(j * 16, 16)] = zeros16

        base_m = sub * stripe_m
        for off in range(0, stripe_m, CH):
            blk = min(CH, stripe_m - off)
            pltpu.sync_copy(rows_vm.at[0, pl.ds(0, blk)],
                            m_sh.at[pl.ds(base_m + off, blk)])
        base_d = sub * stripe_d
        for off in range(0, stripe_d, CH):
            pltpu.sync_copy(s_vm.at[pl.ds(0, CH)], d_sh.at[pl.ds(base_d + off, CH)])
        plsc.subcore_barrier()

        t_w = full + 2 * jnp.where(wid < ex_pairs, 1, 0)

        def cid_of(c):
            return jnp.where(c < full, wid + NW * c,
                             full * NW + wid * 2 + (c - full))

        def start_idx(buf, c):
            off = cid_of(c) * CH
            pltpu.async_copy(ei_hbm.at[0, pl.ds(off, CH)], idx_vm.at[buf, 0], si[buf])
            pltpu.async_copy(ei_hbm.at[1, pl.ds(off, CH)], idx_vm.at[buf, 1], si[buf])
            pltpu.async_copy(ae_hbm.at[pl.ds(off, CH)], ae_vm.at[buf], si[buf])

        def wait_idx(buf, c):
            off = cid_of(c) * CH
            pltpu.make_async_copy(ei_hbm.at[0, pl.ds(off, CH)], idx_vm.at[buf, 0], si[buf]).wait()
            pltpu.make_async_copy(ei_hbm.at[1, pl.ds(off, CH)], idx_vm.at[buf, 1], si[buf]).wait()
            pltpu.make_async_copy(ae_hbm.at[pl.ds(off, CH)], ae_vm.at[buf], si[buf]).wait()

        def start_gather(buf):
            pltpu.async_copy(xs_hbm.at[idx_vm.at[buf, 0]], rows_vm.at[buf], sg[buf])

        def wait_gather(buf):
            pltpu.make_async_copy(xs_hbm.at[idx_vm.at[buf, 0]], rows_vm.at[buf],
                                  sg[buf]).wait()

        start_idx(0, 0)
        wait_idx(0, 0)
        start_gather(0)
        start_idx(1, 1)

        def body(t, carry):
            for b in (0, 1):
                c = 2 * t + b
                nb = 1 - b
                for v in range(8):
                    didx = idx_vm[b, 1, pl.ds(v * 16, 16)]
                    al = plsc.load_gather(ad_vm, [didx]) + ae_vm[b, pl.ds(v * 16, 16)]
                    s_vm[pl.ds(v * 16, 16)] = al
                    dst_vm[pl.ds(v * 16, 16)] = didx
                @pl.when(c + 1 < t_w)
                def _():
                    wait_idx(nb, c + 1)
                    start_gather(nb)
                wait_gather(b)
                @pl.when(c + 2 < t_w)
                def _():
                    start_idx(b, c + 2)
                for v in range(8):
                    ridx = lax.iota(jnp.int32, 16) + v * 16
                    asv = plsc.load_gather(rows_vm.at[b], [ridx, col128])
                    al = s_vm[pl.ds(v * 16, 16)] + asv
                    al = jnp.maximum(al, 0.2 * al)
                    s_vm[pl.ds(v * 16, 16)] = jnp.exp(al)

                def scale(i, carry2):
                    sc = s_vm[pl.ds(i, 16)][0]
                    for j in range(8):
                        rows_vm[b, i, pl.ds(j * 16, 16)] = (
                            rows_vm[b, i, pl.ds(j * 16, 16)] * sc)
                    return carry2

                lax.fori_loop(0, CH, scale, 0)
                pltpu.sync_copy(rows_vm.at[b], m_sh.at[dst_vm], add=True)
                pltpu.sync_copy(s_vm.at[pl.ds(0, CH)], d_sh.at[dst_vm], add=True)
            return carry

        lax.fori_loop(0, t_w // 2, body, 0)

        s_w = (s_chunks - wid + NW - 1) // NW

        def sl_chunk(u, carry):
            cid = wid + NW * u
            nb0 = cid * CH
            nbc = jnp.minimum(nb0, n - CH)
            for v in range(8):
                nodes = nbc + v * 16 + lax.iota(jnp.int32, 16)
                dst_vm[pl.ds(v * 16, 16)] = nodes
            pltpu.sync_copy(lae_hbm.at[pl.ds(nbc, CH)], ae_vm.at[0])
            pltpu.sync_copy(xs_hbm.at[pl.ds(nbc, CH)], rows_vm.at[0])
            for v in range(8):
                nodes = nbc + v * 16 + lax.iota(jnp.int32, 16)
                ridx = lax.iota(jnp.int32, 16) + v * 16
                asv = plsc.load_gather(rows_vm.at[0], [ridx, col128])
                adv = ad_vm[pl.ds(nbc + v * 16, 16)]
                al = asv + adv + ae_vm[0, pl.ds(v * 16, 16)]
                al = jnp.maximum(al, 0.2 * al)
                s = jnp.exp(al)
                s = jnp.where(nodes >= nb0, s, 0.0)
                s_vm[pl.ds(v * 16, 16)] = s

            def scale(i, carry2):
                sc = s_vm[pl.ds(i, 16)][0]
                for j in range(8):
                    rows_vm[0, i, pl.ds(j * 16, 16)] = (
                        rows_vm[0, i, pl.ds(j * 16, 16)] * sc)
                return carry2

            lax.fori_loop(0, CH, scale, 0)
            pltpu.sync_copy(rows_vm.at[0], m_sh.at[dst_vm], add=True)
            pltpu.sync_copy(s_vm.at[pl.ds(0, CH)], d_sh.at[dst_vm], add=True)
            return carry

        lax.fori_loop(0, s_w, sl_chunk, 0)

        plsc.subcore_barrier()
        pltpu.sync_copy(m_sh.at[pl.ds(base_m, stripe_m)],
                        m_hbm.at[core, pl.ds(base_m, stripe_m)])
        pltpu.sync_copy(d_sh.at[pl.ds(base_d, stripe_d)],
                        den_hbm.at[core, pl.ds(base_d, stripe_d)])

    return k(edge_index, ae_e, loop_ae_l, a_d, xse)


def _dense_body(h_ref, w_ref, asrc_ref, adst_ref, xse_ref, ad_ref):
    n = h_ref.shape[0]
    xs = jnp.dot(h_ref[...], w_ref[...], preferred_element_type=jnp.float32)
    a_s = xs @ asrc_ref[...]
    xse_ref[...] = jnp.concatenate(
        [xs, a_s[:, None], jnp.zeros((n, XW - 129), jnp.float32)], axis=1)
    ad_ref[...] = xs @ adst_ref[...]


def _post_body(m_ref, den_ref, bias_ref, bng_ref, bnb_ref, h_ref, hout_ref):
    n = h_ref.shape[0]
    msum = m_ref[0] + m_ref[1]
    den = den_ref[0] + den_ref[1]
    out = msum[:, :128] / (den[:n, None] + 1e-16) + bias_ref[...][None, :]
    m = jnp.mean(out, axis=0)
    v = jnp.mean((out - m[None, :]) ** 2, axis=0)
    out = (out - m[None, :]) * lax.rsqrt(v + 1e-5) * bng_ref[...][None, :] + bnb_ref[...][None, :]
    hout_ref[...] = h_ref[...] + jnp.maximum(out, 0.0)


def _head_body(h_ref, w_ref, b_ref, o_ref):
    g = jnp.mean(h_ref[...], axis=0, keepdims=True)
    o_ref[...] = jnp.maximum(
        jnp.dot(g, w_ref[...], preferred_element_type=jnp.float32) + b_ref[...][None, :],
        0.0)


def _ae_real_body(attr_r_ref, we_ref, ae_ref, *out_refs):
    L = we_ref.shape[0]
    w_all = jnp.sum(we_ref[...] * ae_ref[...][:, None, :], axis=2)
    eye8 = (lax.broadcasted_iota(jnp.int32, (8, 8), 0) ==
            lax.broadcasted_iota(jnp.int32, (8, 8), 1)).astype(jnp.float32)
    for l in range(L):
        wbig = (eye8[:, None, :] * w_all[l][None, :, None]).reshape(128, 8)
        out_refs[l][...] = jnp.dot(attr_r_ref[...], wbig,
                                   preferred_element_type=jnp.float32)


def _loop_ae_body(ssum_ref, deg_ref, we_ref, ae_ref, *out_refs):
    L = we_ref.shape[0]
    n8 = out_refs[0].shape[0]
    w_all = jnp.sum(we_ref[...] * ae_ref[...][:, None, :], axis=2)
    eye8 = (lax.broadcasted_iota(jnp.int32, (8, 8), 0) ==
            lax.broadcasted_iota(jnp.int32, (8, 8), 1)).astype(jnp.float32)
    s8 = ssum_ref[0] + ssum_ref[1]
    inv = 1.0 / jnp.maximum(deg_ref[0, :n8] + deg_ref[1, :n8], 1.0)
    for l in range(L):
        wbig = (eye8[:, None, :] * w_all[l][None, :, None]).reshape(128, 8)
        out_refs[l][...] = jnp.dot(s8, wbig,
                                   preferred_element_type=jnp.float32) * inv


def kernel(x, edge_index, edge_attr, enc_W, enc_b, enc_gamma, enc_beta, W,
           att_src, att_dst, W_edge, att_edge, bias, bn_gamma, bn_beta, out_W, out_b):
    n = x.shape[0]
    E = edge_index.shape[1]
    L = W.shape[0]

    h = pl.pallas_call(
        _encoder_body,
        out_shape=jax.ShapeDtypeStruct((n, enc_W.shape[1]), jnp.float32),
    )(x, enc_W, enc_b, enc_gamma, enc_beta)

    n_pad = NS * CH * _ceil_div(n, NS * CH)
    ssum, deg = _loop_stats_sc(edge_index, edge_attr, n, n_pad)

    attr_r = edge_attr.reshape(E // 8, 128)
    n_blk = 8
    br = E // 8 // n_blk
    ae_parts = pl.pallas_call(
        _ae_real_body,
        grid=(n_blk,),
        in_specs=[pl.BlockSpec((br, 128), lambda i: (i, 0)),
                  pl.BlockSpec(W_edge.shape, lambda i: (0, 0, 0)),
                  pl.BlockSpec(att_edge.shape, lambda i: (0, 0))],
        out_specs=[pl.BlockSpec((br, 8), lambda i: (i, 0))] * L,
        out_shape=[jax.ShapeDtypeStruct((E // 8, 8), jnp.float32)] * L,
    )(attr_r, W_edge, att_edge)
    ae_l = [p.reshape(E) for p in ae_parts]
    loop_ae_l = pl.pallas_call(
        _loop_ae_body,
        out_shape=[jax.ShapeDtypeStruct((n // 8, 8), jnp.float32)] * L,
    )(ssum.reshape(NC, n // 8, 128), deg.reshape(NC, n_pad // 8, 8),
      W_edge, att_edge)
    loop_ae_l = [o.reshape(n) for o in loop_ae_l]

    for l in range(L):
        xse, a_d = pl.pallas_call(
            _dense_body,
            out_shape=(jax.ShapeDtypeStruct((n, XW), jnp.float32),
                       jax.ShapeDtypeStruct((n,), jnp.float32)),
        )(h, W[l], att_src[l], att_dst[l])
        m_parts, den_parts = _edge_pass_sc(edge_index, ae_l[l], loop_ae_l[l],
                                           a_d, xse, n_pad)
        h = pl.pallas_call(
            _post_body,
            out_shape=jax.ShapeDtypeStruct((n, W.shape[2]), jnp.float32),
        )(m_parts, den_parts, bias[l], bn_gamma[l], bn_beta[l], h)

    return pl.pallas_call(
        _head_body,
        out_shape=jax.ShapeDtypeStruct((1, out_W.shape[1]), jnp.float32),
    )(h, out_W, out_b)
